# baseline (speedup 1.0000x reference)
import functools

import jax
import jax.numpy as jnp
from jax import lax
from jax.experimental import pallas as pl
from jax.experimental.pallas import tpu as pltpu
from jax.experimental.pallas import tpu_sc as plsc

B = 16384
K = 32
L = 16

_info = plsc.get_sparse_core_info()
NC, NS = _info.num_cores, _info.num_subcores
NW = NC * NS
BPW = B // NW
IDX_CHUNK = 128
N_IDX = BPW // IDX_CHUNK
N_CHUNKS = BPW // L


def _gmf_body(uid_hbm, iid_hbm, user_hbm, item_hbm, wb_hbm, out_hbm,
              uidx, iidx, urows, irows, outv, wbv, sem_idx, sem_rows):
    wid = lax.axis_index("s") * NC + lax.axis_index("c")
    base = wid * BPW

    idx_copies = []
    for j in range(N_IDX):
        off = base + j * IDX_CHUNK
        idx_copies.append(pltpu.make_async_copy(
            uid_hbm.at[pl.ds(off, IDX_CHUNK)], uidx.at[j], sem_idx))
        idx_copies.append(pltpu.make_async_copy(
            iid_hbm.at[pl.ds(off, IDX_CHUNK)], iidx.at[j], sem_idx))
    for c in idx_copies:
        c.start()
    pltpu.sync_copy(wb_hbm, wbv)
    for c in idx_copies:
        c.wait()

    row_copies = []
    for j in range(N_IDX):
        dst = pl.ds(j * IDX_CHUNK, IDX_CHUNK)
        row_copies.append(pltpu.make_async_copy(
            user_hbm.at[uidx.at[j]], urows.at[dst, :], sem_rows))
        row_copies.append(pltpu.make_async_copy(
            item_hbm.at[iidx.at[j]], irows.at[dst, :], sem_rows))
    for c in row_copies:
        c.start()
    for c in row_copies:
        c.wait()

    def _splat(i):
        f = i + 1
        return plsc.load_gather(
            wbv, [jnp.full((L,), f // L, jnp.int32),
                  jnp.full((L,), f % L, jnp.int32)])

    bias = _splat(K)
    wk = [_splat(k) for k in range(K)]

    def chunk_body(chunk, carry):
        ridx = chunk * L + lax.iota(jnp.int32, L)
        acc0 = bias
        acc1 = jnp.zeros((L,), jnp.float32)
        for k in range(K):
            kidx = jnp.full((L,), k, jnp.int32)
            cu = plsc.load_gather(urows, [ridx, kidx])
            ci = plsc.load_gather(irows, [ridx, kidx])
            term = cu * ci * wk[k]
            if k % 2 == 0:
                acc0 = acc0 + term
            else:
                acc1 = acc1 + term
        acc = acc0 + acc1
        outv[pl.ds(chunk * L, L)] = 1.0 / (1.0 + jnp.exp(-acc))
        return carry

    lax.fori_loop(0, N_CHUNKS, chunk_body, 0)
    pltpu.sync_copy(outv, out_hbm.at[pl.ds(base, BPW)])


@jax.jit
def kernel(uid, iid, user_mat, item_mat, affine_w, affine_b):
    mesh = plsc.VectorSubcoreMesh(core_axis_name="c", subcore_axis_name="s")
    f = pl.kernel(
        _gmf_body,
        out_type=jax.ShapeDtypeStruct((B,), jnp.float32),
        mesh=mesh,
        compiler_params=pltpu.CompilerParams(
            needs_layout_passes=False, use_tc_tiling_on_sc=False),
        scratch_types=[
            pltpu.VMEM((N_IDX, IDX_CHUNK), jnp.int32),
            pltpu.VMEM((N_IDX, IDX_CHUNK), jnp.int32),
            pltpu.VMEM((BPW, K), jnp.float32),
            pltpu.VMEM((BPW, K), jnp.float32),
            pltpu.VMEM((BPW,), jnp.float32),
            pltpu.VMEM((8, L), jnp.float32),
            pltpu.SemaphoreType.DMA,
            pltpu.SemaphoreType.DMA,
        ],
    )
    wb = jnp.zeros((8 * L,), jnp.float32)
    wb = wb.at[1:K + 1].set(affine_w[0]).at[K + 1].set(affine_b[0]).reshape(8, L)
    return f(uid, iid, user_mat, item_mat, wb)


# speedup vs baseline: 3.2476x; 3.2476x over previous
import functools

import jax
import jax.numpy as jnp
from jax import lax
from jax.experimental import pallas as pl
from jax.experimental.pallas import tpu as pltpu
from jax.experimental.pallas import tpu_sc as plsc

B = 16384
K = 32
L = 16
TW = 128

_info = plsc.get_sparse_core_info()
NC, NS = _info.num_cores, _info.num_subcores
NW = NC * NS
BPW = B // NW
WAVE = 4
N_GROUPS = BPW // L
N_CHUNKS = BPW // L


def _gmf_body(uid_hbm, iid_hbm, user_t, item_t, wb_hbm, out_hbm,
              uidx, iidx, ublk, iblk, grows, girows, outv, wbv,
              sem_idx, sem_a, sem_b):
    wid = lax.axis_index("s") * NC + lax.axis_index("c")
    base = wid * BPW

    cp_u = pltpu.make_async_copy(uid_hbm.at[pl.ds(base, BPW)], uidx, sem_idx)
    cp_i = pltpu.make_async_copy(iid_hbm.at[pl.ds(base, BPW)], iidx, sem_idx)
    cp_u.start()
    cp_i.start()
    pltpu.sync_copy(wb_hbm, wbv)
    cp_u.wait()
    cp_i.wait()

    iota = lax.iota(jnp.int32, L)
    lo = iota % jnp.int32(L)
    hi = iota + jnp.int32(L)

    def _fetch_wave(uvec, ivec, w, ring, sem):
        for l in range(WAVE):
            ru = uvec[w * WAVE + l]
            ri = ivec[w * WAVE + l]
            bu = pl.multiple_of((ru // TW) * TW, TW)
            bi = pl.multiple_of((ri // TW) * TW, TW)
            pltpu.make_async_copy(
                user_t.at[:, pl.ds(bu, TW)], ublk.at[ring, l], sem).start()
            pltpu.make_async_copy(
                item_t.at[:, pl.ds(bi, TW)], iblk.at[ring, l], sem).start()

    def _drain_wave(ring, sem):
        dummy = user_t.at[pl.ds(0, K), pl.ds(0, TW)]
        for l in range(WAVE):
            pltpu.make_async_copy(dummy, ublk.at[ring, l], sem).wait()
            pltpu.make_async_copy(dummy, iblk.at[ring, l], sem).wait()

    def _extract_wave(uvec, ivec, w, ring):
        for l in range(WAVE):
            p = w * WAVE + l
            cu = jnp.full((L,), uvec[w * WAVE + l] % TW, jnp.int32)
            ci = jnp.full((L,), ivec[w * WAVE + l] % TW, jnp.int32)
            grows[p, pl.ds(0, L)] = plsc.load_gather(ublk.at[ring, l], [lo, cu])
            grows[p, pl.ds(L, L)] = plsc.load_gather(ublk.at[ring, l], [hi, cu])
            girows[p, pl.ds(0, L)] = plsc.load_gather(iblk.at[ring, l], [lo, ci])
            girows[p, pl.ds(L, L)] = plsc.load_gather(iblk.at[ring, l], [hi, ci])

    n_waves = L // WAVE

    w0 = wbv[0, pl.ds(0, L)]
    w1 = wbv[0, pl.ds(L, L)]
    bvec = wbv[0, pl.ds(K, L)]
    wk = [jnp.full((L,), (w0 if k < L else w1)[k % L], jnp.float32)
          for k in range(K)]
    bias = jnp.full((L,), bvec[0], jnp.float32)

    def group_body(g, carry):
        uvec = uidx[pl.ds(g * L, L)]
        ivec = iidx[pl.ds(g * L, L)]
        _fetch_wave(uvec, ivec, 0, 0, sem_a)
        for w in range(n_waves):
            ring, sem = w % 2, (sem_a if w % 2 == 0 else sem_b)
            if w + 1 < n_waves:
                nring = (w + 1) % 2
                nsem = sem_a if (w + 1) % 2 == 0 else sem_b
                _fetch_wave(uvec, ivec, w + 1, nring, nsem)
            _drain_wave(ring, sem)
            _extract_wave(uvec, ivec, w, ring)
        acc0 = bias
        acc1 = jnp.zeros((L,), jnp.float32)
        for k in range(K):
            kidx = jnp.full((L,), k, jnp.int32)
            term = (plsc.load_gather(grows, [iota, kidx])
                    * plsc.load_gather(girows, [iota, kidx]) * wk[k])
            if k % 2 == 0:
                acc0 = acc0 + term
            else:
                acc1 = acc1 + term
        acc = acc0 + acc1
        outv[pl.ds(g * L, L)] = 1.0 / (1.0 + jnp.exp(-acc))
        return carry

    lax.fori_loop(0, N_GROUPS, group_body, 0)
    pltpu.sync_copy(outv, out_hbm.at[pl.ds(base, BPW)])


@jax.jit
def kernel(uid, iid, user_mat, item_mat, affine_w, affine_b):
    mesh = plsc.VectorSubcoreMesh(core_axis_name="c", subcore_axis_name="s")
    f = pl.kernel(
        _gmf_body,
        out_type=jax.ShapeDtypeStruct((B,), jnp.float32),
        mesh=mesh,
        compiler_params=pltpu.CompilerParams(needs_layout_passes=False),
        scratch_types=[
            pltpu.VMEM((BPW,), jnp.int32),
            pltpu.VMEM((BPW,), jnp.int32),
            pltpu.VMEM((2, WAVE, K, TW), jnp.float32),
            pltpu.VMEM((2, WAVE, K, TW), jnp.float32),
            pltpu.VMEM((L, K), jnp.float32),
            pltpu.VMEM((L, K), jnp.float32),
            pltpu.VMEM((BPW,), jnp.float32),
            pltpu.VMEM((1, 128), jnp.float32),
            pltpu.SemaphoreType.DMA,
            pltpu.SemaphoreType.DMA,
            pltpu.SemaphoreType.DMA,
        ],
    )
    wb = jnp.zeros((128,), jnp.float32)
    wb = wb.at[:K].set(affine_w[0]).at[K].set(affine_b[0]).reshape(1, 128)
    return f(uid, iid, user_mat.T, item_mat.T, wb)
